# Initial kernel scaffold; baseline (speedup 1.0000x reference)
import jax
import jax.numpy as jnp
from jax.experimental import pallas as pl


def kernel(x, edge_index, batch, params):
    raise NotImplementedError("write your pallas kernel here")


# speedup vs baseline: 3.6499x; 3.6499x over previous
import functools

import jax
import jax.numpy as jnp
from jax import lax
from jax.experimental import pallas as pl
from jax.experimental.pallas import tpu as pltpu
from jax.experimental.pallas import tpu_sc as plsc

_N = 10000
_E = 640000
_D = 128
_H = 64
_G = 64

_NW = 32
_CH = 128
_SLAB = 32
_NSL = 5
_NCH = _SLAB * _NSL
_EPW = _CH * _NCH
_EPAD = _NW * _EPW
_RPT = 640
_NP = 16 * _RPT
_DUMMY = _N

_NB = 10
_R = _N // _NB
_EPS = 1e-5
_PREC = lax.Precision.HIGHEST


def _seg_sum_sc(dh):
    mesh = plsc.VectorSubcoreMesh(core_axis_name="c", subcore_axis_name="s")

    @functools.partial(
        pl.kernel,
        out_type=jax.ShapeDtypeStruct((2, _NP, dh), jnp.float32),
        mesh=mesh,
        scratch_types=[
            pltpu.VMEM((_SLAB, _CH), jnp.int32),
            pltpu.VMEM((_SLAB, _CH), jnp.int32),
            pltpu.VMEM((_CH, dh), jnp.float32),
            pltpu.VMEM_SHARED((_NP, dh), jnp.float32),
            pltpu.SemaphoreType.DMA,
        ],
    )
    def k(h_hbm, src_hbm, dst_hbm, z_hbm, out_hbm, src_v, dst_v, rows_v,
          agg_sh, sem):
        c = lax.axis_index("c")
        s = lax.axis_index("s")
        wid = s * 2 + c
        pltpu.sync_copy(z_hbm, agg_sh.at[pl.ds(s * _RPT, _RPT)])
        plsc.subcore_barrier()

        def slab(t, carry):
            pltpu.sync_copy(src_hbm.at[wid, pl.ds(t * _SLAB, _SLAB)], src_v)
            pltpu.sync_copy(dst_hbm.at[wid, pl.ds(t * _SLAB, _SLAB)], dst_v)

            def chunk(j, carry2):
                pltpu.async_copy(h_hbm.at[src_v.at[j]], rows_v, sem).wait()
                pltpu.sync_copy(rows_v, agg_sh.at[dst_v.at[j]], add=True)
                return carry2

            return lax.fori_loop(0, _SLAB, chunk, carry)

        lax.fori_loop(0, _NSL, slab, 0)
        plsc.subcore_barrier()
        pltpu.sync_copy(agg_sh.at[pl.ds(s * _RPT, _RPT)],
                        out_hbm.at[c, pl.ds(s * _RPT, _RPT)])

    return k


def _mm_stats(base, a0, a1, W, b, din):
    def body(base_ref, a0_ref, a1_ref, W_ref, b_ref, hp_ref, st_ref):
        i = pl.program_id(0)
        u = base_ref[...] + a0_ref[...] + a1_ref[...]
        hp = jnp.dot(u, W_ref[...], precision=_PREC,
                     preferred_element_type=jnp.float32) + b_ref[...]
        hp_ref[...] = hp

        @pl.when(i == 0)
        def _():
            st_ref[...] = jnp.zeros_like(st_ref)

        st_ref[0:1, :] += jnp.sum(hp, axis=0, keepdims=True)
        st_ref[1:2, :] += jnp.sum(hp * hp, axis=0, keepdims=True)

    return pl.pallas_call(
        body,
        grid=(_NB,),
        in_specs=[
            pl.BlockSpec((_R, din), lambda i: (i, 0)),
            pl.BlockSpec((_R, din), lambda i: (i, 0)),
            pl.BlockSpec((_R, din), lambda i: (i, 0)),
            pl.BlockSpec((din, _H), lambda i: (0, 0)),
            pl.BlockSpec((1, _H), lambda i: (0, 0)),
        ],
        out_specs=[
            pl.BlockSpec((_R, _H), lambda i: (i, 0)),
            pl.BlockSpec((8, _H), lambda i: (0, 0)),
        ],
        out_shape=[
            jax.ShapeDtypeStruct((_N, _H), jnp.float32),
            jax.ShapeDtypeStruct((8, _H), jnp.float32),
        ],
    )(base, a0, a1, W, b)


def _bn_scale_shift(st_ref, gamma, beta):
    mean = st_ref[0:1, :] * (1.0 / _N)
    var = st_ref[1:2, :] * (1.0 / _N) - mean * mean
    scale = gamma * lax.rsqrt(var + _EPS)
    shift = beta - mean * scale
    return scale, shift


def _bn_mm(hp, st, gamma, beta, W, b):
    def body(hp_ref, st_ref, g_ref, be_ref, W_ref, b_ref, o_ref, st2_ref):
        i = pl.program_id(0)
        scale, shift = _bn_scale_shift(st_ref, g_ref[...], be_ref[...])
        r = jnp.maximum(hp_ref[...] * scale + shift, 0.0)
        hp2 = jnp.dot(r, W_ref[...], precision=_PREC,
                      preferred_element_type=jnp.float32) + b_ref[...]
        o_ref[...] = hp2

        @pl.when(i == 0)
        def _():
            st2_ref[...] = jnp.zeros_like(st2_ref)

        st2_ref[0:1, :] += jnp.sum(hp2, axis=0, keepdims=True)
        st2_ref[1:2, :] += jnp.sum(hp2 * hp2, axis=0, keepdims=True)

    return pl.pallas_call(
        body,
        grid=(_NB,),
        in_specs=[
            pl.BlockSpec((_R, _H), lambda i: (i, 0)),
            pl.BlockSpec((8, _H), lambda i: (0, 0)),
            pl.BlockSpec((1, _H), lambda i: (0, 0)),
            pl.BlockSpec((1, _H), lambda i: (0, 0)),
            pl.BlockSpec((_H, _H), lambda i: (0, 0)),
            pl.BlockSpec((1, _H), lambda i: (0, 0)),
        ],
        out_specs=[
            pl.BlockSpec((_R, _H), lambda i: (i, 0)),
            pl.BlockSpec((8, _H), lambda i: (0, 0)),
        ],
        out_shape=[
            jax.ShapeDtypeStruct((_N, _H), jnp.float32),
            jax.ShapeDtypeStruct((8, _H), jnp.float32),
        ],
    )(hp, st, gamma, beta, W, b)


def _bn_res(hp, st, gamma, beta, res):
    has_res = res is not None

    def body(*refs):
        if has_res:
            hp_ref, st_ref, g_ref, be_ref, r_ref, o_ref = refs
        else:
            hp_ref, st_ref, g_ref, be_ref, o_ref = refs
        scale, shift = _bn_scale_shift(st_ref, g_ref[...], be_ref[...])
        h = jnp.maximum(hp_ref[...] * scale + shift, 0.0)
        if has_res:
            h = h + r_ref[:, : _H]
        o_ref[:, : _H] = h
        o_ref[:, _H :] = jnp.zeros((_R, _D - _H), jnp.float32)

    in_specs = [
        pl.BlockSpec((_R, _H), lambda i: (i, 0)),
        pl.BlockSpec((8, _H), lambda i: (0, 0)),
        pl.BlockSpec((1, _H), lambda i: (0, 0)),
        pl.BlockSpec((1, _H), lambda i: (0, 0)),
    ]
    args = [hp, st, gamma, beta]
    if has_res:
        in_specs.append(pl.BlockSpec((_R, _D), lambda i: (i, 0)))
        args.append(res)

    return pl.pallas_call(
        body,
        grid=(_NB,),
        in_specs=in_specs,
        out_specs=pl.BlockSpec((_R, _D), lambda i: (i, 0)),
        out_shape=jax.ShapeDtypeStruct((_N, _D), jnp.float32),
    )(*args)


def _pool(h, batch_f):
    def body(b_ref, h_ref, g_ref):
        i = pl.program_id(0)
        labels = b_ref[0, 0, :]
        seg = lax.broadcasted_iota(jnp.int32, (_R, _G), 1).astype(jnp.float32)
        onehot = jnp.where(labels[:, None] == seg, 1.0, 0.0)
        part = lax.dot_general(onehot, h_ref[:, : _H], (((0,), (0,)), ((), ())),
                               precision=_PREC,
                               preferred_element_type=jnp.float32)

        @pl.when(i == 0)
        def _():
            g_ref[...] = jnp.zeros_like(g_ref)

        g_ref[...] += part

    return pl.pallas_call(
        body,
        grid=(_NB,),
        in_specs=[
            pl.BlockSpec((1, 1, _R), lambda i: (i, 0, 0)),
            pl.BlockSpec((_R, _D), lambda i: (i, 0)),
        ],
        out_specs=pl.BlockSpec((_G, _H), lambda i: (0, 0)),
        out_shape=jax.ShapeDtypeStruct((_G, _H), jnp.float32),
    )(batch_f, h)


def _heads(g, hp):
    names = ['head_color', 'head_size', 'head_ground', 'head_struct']
    douts = [16, 16, 8, 32]

    def body(*refs):
        g_ref = refs[0]
        outs = refs[1 + 4 * len(names):]
        gv = g_ref[...]
        for n in range(len(names)):
            W1, b1, W2, b2 = refs[1 + 4 * n:1 + 4 * (n + 1)]
            r = jnp.maximum(
                jnp.dot(gv, W1[...], precision=_PREC,
                        preferred_element_type=jnp.float32) + b1[...], 0.0)
            z = jnp.dot(r, W2[...], precision=_PREC,
                        preferred_element_type=jnp.float32) + b2[...]
            nrm = jnp.sqrt(jnp.sum(z * z, axis=1, keepdims=True))
            outs[n][...] = z / jnp.maximum(nrm, 1e-12)

    args = [g]
    for n in names:
        p = hp[n]
        args += [p['W1'], p['b1'].reshape(1, -1), p['W2'],
                 p['b2'].reshape(1, -1)]

    return pl.pallas_call(
        body,
        out_shape=[jax.ShapeDtypeStruct((_G, d), jnp.float32) for d in douts],
    )(*args)


def kernel(x, edge_index, batch, params):
    src = edge_index[0]
    dst = edge_index[1]
    pad_src = jnp.concatenate(
        [src, jnp.zeros((_EPAD - _E,), jnp.int32)]).reshape(_NW, _NCH, _CH)
    pad_dst = jnp.concatenate(
        [dst, jnp.full((_EPAD - _E,), _DUMMY, jnp.int32)]).reshape(
            _NW, _NCH, _CH)
    zblk = jnp.zeros((_RPT, _D), jnp.float32)
    batch_f = batch.astype(jnp.float32).reshape(_NB, 1, _R)

    seg = _seg_sum_sc(_D)

    def gin(hin, W1, p, res):
        agg = seg(hin, pad_src, pad_dst, zblk)
        a0 = lax.slice(agg, (0, 0, 0), (1, _N, _D)).reshape(_N, _D)
        a1 = lax.slice(agg, (1, 0, 0), (2, _N, _D)).reshape(_N, _D)
        hp, st = _mm_stats(hin, a0, a1, W1, p['b1'].reshape(1, -1), _D)
        hp2, st2 = _bn_mm(hp, st, p['g1'].reshape(1, -1),
                          p['be1'].reshape(1, -1), p['W2'],
                          p['b2'].reshape(1, -1))
        return _bn_res(hp2, st2, p['g2'].reshape(1, -1),
                       p['be2'].reshape(1, -1), res)

    pad_w = lambda W: jnp.concatenate(
        [W, jnp.zeros((_D - _H, _H), jnp.float32)], axis=0)
    h1 = gin(x, params['conv1']['W1'], params['conv1'], None)
    h2 = gin(h1, pad_w(params['conv2']['W1']), params['conv2'], h1)
    h3 = gin(h2, pad_w(params['conv3']['W1']), params['conv3'], h2)
    g = _pool(h3, batch_f)
    return _heads(g, params)


# speedup vs baseline: 4.1016x; 1.1238x over previous
import functools

import jax
import jax.numpy as jnp
from jax import lax
from jax.experimental import pallas as pl
from jax.experimental.pallas import tpu as pltpu
from jax.experimental.pallas import tpu_sc as plsc

_N = 10000
_E = 640000
_D = 128
_H = 64
_G = 64

_NW = 32
_CH = 128
_SLAB = 32
_NSL = 5
_NCH = _SLAB * _NSL
_EPW = _CH * _NCH
_EPAD = _NW * _EPW
_RPT = 640
_NP = 16 * _RPT
_DUMMY = _N

_NB = 10
_R = _N // _NB
_EPS = 1e-5
_PREC = lax.Precision.HIGHEST


def _seg_sum_sc(dh):
    mesh = plsc.VectorSubcoreMesh(core_axis_name="c", subcore_axis_name="s")

    @functools.partial(
        pl.kernel,
        out_type=jax.ShapeDtypeStruct((2, _NP, dh), jnp.float32),
        mesh=mesh,
        scratch_types=[
            pltpu.VMEM((_SLAB, _CH), jnp.int32),
            pltpu.VMEM((_SLAB, _CH), jnp.int32),
            pltpu.VMEM((_CH, dh), jnp.float32),
            pltpu.VMEM((_CH, dh), jnp.float32),
            pltpu.VMEM_SHARED((_NP, dh), jnp.float32),
            pltpu.SemaphoreType.DMA,
            pltpu.SemaphoreType.DMA,
            pltpu.SemaphoreType.DMA,
            pltpu.SemaphoreType.DMA,
        ],
    )
    def k(h_hbm, src_hbm, dst_hbm, z_hbm, out_hbm, src_v, dst_v, rows_a,
          rows_b, agg_sh, sga, sgb, ssa, ssb):
        c = lax.axis_index("c")
        s = lax.axis_index("s")
        wid = s * 2 + c
        pltpu.sync_copy(z_hbm, agg_sh.at[pl.ds(s * _RPT, _RPT)])
        plsc.subcore_barrier()

        def slab(t, carry):
            pltpu.sync_copy(src_hbm.at[wid, pl.ds(t * _SLAB, _SLAB)], src_v)
            pltpu.sync_copy(dst_hbm.at[wid, pl.ds(t * _SLAB, _SLAB)], dst_v)

            def pair(p, carry2):
                ga = pltpu.async_copy(h_hbm.at[src_v.at[2 * p]], rows_a, sga)
                gb = pltpu.async_copy(h_hbm.at[src_v.at[2 * p + 1]], rows_b,
                                      sgb)
                ga.wait()
                sa = pltpu.async_copy(rows_a, agg_sh.at[dst_v.at[2 * p]],
                                      ssa, add=True)
                gb.wait()
                sb = pltpu.async_copy(rows_b, agg_sh.at[dst_v.at[2 * p + 1]],
                                      ssb, add=True)
                sa.wait()
                sb.wait()
                return carry2

            return lax.fori_loop(0, _SLAB // 2, pair, carry)

        lax.fori_loop(0, _NSL, slab, 0)
        plsc.subcore_barrier()
        pltpu.sync_copy(agg_sh.at[pl.ds(s * _RPT, _RPT)],
                        out_hbm.at[c, pl.ds(s * _RPT, _RPT)])

    return k


def _mm_stats(base, a0, a1, W, b, din):
    def body(base_ref, a0_ref, a1_ref, W_ref, b_ref, hp_ref, st_ref):
        i = pl.program_id(0)
        u = base_ref[...] + a0_ref[...] + a1_ref[...]
        hp = jnp.dot(u, W_ref[...], precision=_PREC,
                     preferred_element_type=jnp.float32) + b_ref[...]
        hp_ref[...] = hp

        @pl.when(i == 0)
        def _():
            st_ref[...] = jnp.zeros_like(st_ref)

        st_ref[0:1, :] += jnp.sum(hp, axis=0, keepdims=True)
        st_ref[1:2, :] += jnp.sum(hp * hp, axis=0, keepdims=True)

    return pl.pallas_call(
        body,
        grid=(_NB,),
        in_specs=[
            pl.BlockSpec((_R, din), lambda i: (i, 0)),
            pl.BlockSpec((_R, din), lambda i: (i, 0)),
            pl.BlockSpec((_R, din), lambda i: (i, 0)),
            pl.BlockSpec((din, _H), lambda i: (0, 0)),
            pl.BlockSpec((1, _H), lambda i: (0, 0)),
        ],
        out_specs=[
            pl.BlockSpec((_R, _H), lambda i: (i, 0)),
            pl.BlockSpec((8, _H), lambda i: (0, 0)),
        ],
        out_shape=[
            jax.ShapeDtypeStruct((_N, _H), jnp.float32),
            jax.ShapeDtypeStruct((8, _H), jnp.float32),
        ],
    )(base, a0, a1, W, b)


def _bn_scale_shift(st_ref, gamma, beta):
    mean = st_ref[0:1, :] * (1.0 / _N)
    var = st_ref[1:2, :] * (1.0 / _N) - mean * mean
    scale = gamma * lax.rsqrt(var + _EPS)
    shift = beta - mean * scale
    return scale, shift


def _bn_mm(hp, st, gamma, beta, W, b):
    def body(hp_ref, st_ref, g_ref, be_ref, W_ref, b_ref, o_ref, st2_ref):
        i = pl.program_id(0)
        scale, shift = _bn_scale_shift(st_ref, g_ref[...], be_ref[...])
        r = jnp.maximum(hp_ref[...] * scale + shift, 0.0)
        hp2 = jnp.dot(r, W_ref[...], precision=_PREC,
                      preferred_element_type=jnp.float32) + b_ref[...]
        o_ref[...] = hp2

        @pl.when(i == 0)
        def _():
            st2_ref[...] = jnp.zeros_like(st2_ref)

        st2_ref[0:1, :] += jnp.sum(hp2, axis=0, keepdims=True)
        st2_ref[1:2, :] += jnp.sum(hp2 * hp2, axis=0, keepdims=True)

    return pl.pallas_call(
        body,
        grid=(_NB,),
        in_specs=[
            pl.BlockSpec((_R, _H), lambda i: (i, 0)),
            pl.BlockSpec((8, _H), lambda i: (0, 0)),
            pl.BlockSpec((1, _H), lambda i: (0, 0)),
            pl.BlockSpec((1, _H), lambda i: (0, 0)),
            pl.BlockSpec((_H, _H), lambda i: (0, 0)),
            pl.BlockSpec((1, _H), lambda i: (0, 0)),
        ],
        out_specs=[
            pl.BlockSpec((_R, _H), lambda i: (i, 0)),
            pl.BlockSpec((8, _H), lambda i: (0, 0)),
        ],
        out_shape=[
            jax.ShapeDtypeStruct((_N, _H), jnp.float32),
            jax.ShapeDtypeStruct((8, _H), jnp.float32),
        ],
    )(hp, st, gamma, beta, W, b)


def _bn_res(hp, st, gamma, beta, res):
    has_res = res is not None

    def body(*refs):
        if has_res:
            hp_ref, st_ref, g_ref, be_ref, r_ref, o_ref = refs
        else:
            hp_ref, st_ref, g_ref, be_ref, o_ref = refs
        scale, shift = _bn_scale_shift(st_ref, g_ref[...], be_ref[...])
        h = jnp.maximum(hp_ref[...] * scale + shift, 0.0)
        if has_res:
            h = h + r_ref[:, : _H]
        o_ref[:, : _H] = h
        o_ref[:, _H :] = jnp.zeros((_R, _D - _H), jnp.float32)

    in_specs = [
        pl.BlockSpec((_R, _H), lambda i: (i, 0)),
        pl.BlockSpec((8, _H), lambda i: (0, 0)),
        pl.BlockSpec((1, _H), lambda i: (0, 0)),
        pl.BlockSpec((1, _H), lambda i: (0, 0)),
    ]
    args = [hp, st, gamma, beta]
    if has_res:
        in_specs.append(pl.BlockSpec((_R, _D), lambda i: (i, 0)))
        args.append(res)

    return pl.pallas_call(
        body,
        grid=(_NB,),
        in_specs=in_specs,
        out_specs=pl.BlockSpec((_R, _D), lambda i: (i, 0)),
        out_shape=jax.ShapeDtypeStruct((_N, _D), jnp.float32),
    )(*args)


def _pool(h, batch_f):
    def body(b_ref, h_ref, g_ref):
        i = pl.program_id(0)
        labels = b_ref[0, 0, :]
        seg = lax.broadcasted_iota(jnp.int32, (_R, _G), 1).astype(jnp.float32)
        onehot = jnp.where(labels[:, None] == seg, 1.0, 0.0)
        part = lax.dot_general(onehot, h_ref[:, : _H], (((0,), (0,)), ((), ())),
                               precision=_PREC,
                               preferred_element_type=jnp.float32)

        @pl.when(i == 0)
        def _():
            g_ref[...] = jnp.zeros_like(g_ref)

        g_ref[...] += part

    return pl.pallas_call(
        body,
        grid=(_NB,),
        in_specs=[
            pl.BlockSpec((1, 1, _R), lambda i: (i, 0, 0)),
            pl.BlockSpec((_R, _D), lambda i: (i, 0)),
        ],
        out_specs=pl.BlockSpec((_G, _H), lambda i: (0, 0)),
        out_shape=jax.ShapeDtypeStruct((_G, _H), jnp.float32),
    )(batch_f, h)


def _heads(g, hp):
    names = ['head_color', 'head_size', 'head_ground', 'head_struct']
    douts = [16, 16, 8, 32]

    def body(*refs):
        g_ref = refs[0]
        outs = refs[1 + 4 * len(names):]
        gv = g_ref[...]
        for n in range(len(names)):
            W1, b1, W2, b2 = refs[1 + 4 * n:1 + 4 * (n + 1)]
            r = jnp.maximum(
                jnp.dot(gv, W1[...], precision=_PREC,
                        preferred_element_type=jnp.float32) + b1[...], 0.0)
            z = jnp.dot(r, W2[...], precision=_PREC,
                        preferred_element_type=jnp.float32) + b2[...]
            nrm = jnp.sqrt(jnp.sum(z * z, axis=1, keepdims=True))
            outs[n][...] = z / jnp.maximum(nrm, 1e-12)

    args = [g]
    for n in names:
        p = hp[n]
        args += [p['W1'], p['b1'].reshape(1, -1), p['W2'],
                 p['b2'].reshape(1, -1)]

    return pl.pallas_call(
        body,
        out_shape=[jax.ShapeDtypeStruct((_G, d), jnp.float32) for d in douts],
    )(*args)


def kernel(x, edge_index, batch, params):
    src = edge_index[0]
    dst = edge_index[1]
    npad = _EPW - _E // _NW
    dummy_dst = jnp.broadcast_to(
        _DUMMY + (jnp.arange(npad, dtype=jnp.int32) % (_NP - _N)),
        (_NW, npad))
    pad_src = jnp.concatenate(
        [src.reshape(_NW, _E // _NW),
         jnp.zeros((_NW, npad), jnp.int32)], axis=1).reshape(_NW, _NCH, _CH)
    pad_dst = jnp.concatenate(
        [dst.reshape(_NW, _E // _NW), dummy_dst],
        axis=1).reshape(_NW, _NCH, _CH)
    zblk = jnp.zeros((_RPT, _D), jnp.float32)
    batch_f = batch.astype(jnp.float32).reshape(_NB, 1, _R)

    seg = _seg_sum_sc(_D)

    def gin(hin, W1, p, res):
        agg = seg(hin, pad_src, pad_dst, zblk)
        a0 = lax.slice(agg, (0, 0, 0), (1, _N, _D)).reshape(_N, _D)
        a1 = lax.slice(agg, (1, 0, 0), (2, _N, _D)).reshape(_N, _D)
        hp, st = _mm_stats(hin, a0, a1, W1, p['b1'].reshape(1, -1), _D)
        hp2, st2 = _bn_mm(hp, st, p['g1'].reshape(1, -1),
                          p['be1'].reshape(1, -1), p['W2'],
                          p['b2'].reshape(1, -1))
        return _bn_res(hp2, st2, p['g2'].reshape(1, -1),
                       p['be2'].reshape(1, -1), res)

    pad_w = lambda W: jnp.concatenate(
        [W, jnp.zeros((_D - _H, _H), jnp.float32)], axis=0)
    h1 = gin(x, params['conv1']['W1'], params['conv1'], None)
    h2 = gin(h1, pad_w(params['conv2']['W1']), params['conv2'], h1)
    h3 = gin(h2, pad_w(params['conv3']['W1']), params['conv3'], h2)
    g = _pool(h3, batch_f)
    return _heads(g, params)


# speedup vs baseline: 5.7110x; 1.3924x over previous
import functools

import jax
import jax.numpy as jnp
from jax import lax
from jax.experimental import pallas as pl
from jax.experimental.pallas import tpu as pltpu
from jax.experimental.pallas import tpu_sc as plsc

_N = 10000
_E = 640000
_D = 128
_H = 64
_G = 64

_NW = 32
_CH = 128
_SLAB = 32
_NSL = 5
_NCH = _SLAB * _NSL
_EPW = _CH * _NCH
_EPAD = _NW * _EPW
_RPT = 640
_NP = 16 * _RPT
_DUMMY = _N

_NB = 10
_R = _N // _NB
_EPS = 1e-5
_PREC = lax.Precision.HIGHEST


def _seg_sum_sc(dh):
    mesh = plsc.VectorSubcoreMesh(core_axis_name="c", subcore_axis_name="s")

    @functools.partial(
        pl.kernel,
        out_type=jax.ShapeDtypeStruct((2, _NP, dh), jnp.float32),
        mesh=mesh,
        compiler_params=pltpu.CompilerParams(use_tc_tiling_on_sc=False),
        scratch_types=[
            pltpu.VMEM((_SLAB, _CH), jnp.int32),
            pltpu.VMEM((_SLAB, _CH), jnp.int32),
            pltpu.VMEM((_CH, dh), jnp.float32),
            pltpu.VMEM((_CH, dh), jnp.float32),
            pltpu.VMEM_SHARED((_NP, dh), jnp.float32),
            pltpu.SemaphoreType.DMA,
            pltpu.SemaphoreType.DMA,
            pltpu.SemaphoreType.DMA,
            pltpu.SemaphoreType.DMA,
        ],
    )
    def k(h_hbm, src_hbm, dst_hbm, z_hbm, out_hbm, src_v, dst_v, rows_a,
          rows_b, agg_sh, sga, sgb, ssa, ssb):
        c = lax.axis_index("c")
        s = lax.axis_index("s")
        wid = s * 2 + c
        pltpu.sync_copy(z_hbm, agg_sh.at[pl.ds(s * _RPT, _RPT)])
        plsc.subcore_barrier()

        def slab(t, carry):
            pltpu.sync_copy(src_hbm.at[wid, pl.ds(t * _SLAB, _SLAB)], src_v)
            pltpu.sync_copy(dst_hbm.at[wid, pl.ds(t * _SLAB, _SLAB)], dst_v)

            def pair(p, carry2):
                ga = pltpu.async_copy(h_hbm.at[src_v.at[2 * p]], rows_a, sga)
                gb = pltpu.async_copy(h_hbm.at[src_v.at[2 * p + 1]], rows_b,
                                      sgb)
                ga.wait()
                sa = pltpu.async_copy(rows_a, agg_sh.at[dst_v.at[2 * p]],
                                      ssa, add=True)
                gb.wait()
                sb = pltpu.async_copy(rows_b, agg_sh.at[dst_v.at[2 * p + 1]],
                                      ssb, add=True)
                sa.wait()
                sb.wait()
                return carry2

            return lax.fori_loop(0, _SLAB // 2, pair, carry)

        lax.fori_loop(0, _NSL, slab, 0)
        plsc.subcore_barrier()
        pltpu.sync_copy(agg_sh.at[pl.ds(s * _RPT, _RPT)],
                        out_hbm.at[c, pl.ds(s * _RPT, _RPT)])

    return k


def _mm_stats(base, a0, a1, W, b, din):
    def body(base_ref, a0_ref, a1_ref, W_ref, b_ref, hp_ref, st_ref):
        i = pl.program_id(0)
        u = base_ref[...] + a0_ref[...] + a1_ref[...]
        hp = jnp.dot(u, W_ref[...], precision=_PREC,
                     preferred_element_type=jnp.float32) + b_ref[...]
        hp_ref[...] = hp

        @pl.when(i == 0)
        def _():
            st_ref[...] = jnp.zeros_like(st_ref)

        st_ref[0:1, :] += jnp.sum(hp, axis=0, keepdims=True)
        st_ref[1:2, :] += jnp.sum(hp * hp, axis=0, keepdims=True)

    return pl.pallas_call(
        body,
        grid=(_NB,),
        in_specs=[
            pl.BlockSpec((_R, din), lambda i: (i, 0)),
            pl.BlockSpec((_R, din), lambda i: (i, 0)),
            pl.BlockSpec((_R, din), lambda i: (i, 0)),
            pl.BlockSpec((din, _H), lambda i: (0, 0)),
            pl.BlockSpec((1, _H), lambda i: (0, 0)),
        ],
        out_specs=[
            pl.BlockSpec((_R, _H), lambda i: (i, 0)),
            pl.BlockSpec((8, _H), lambda i: (0, 0)),
        ],
        out_shape=[
            jax.ShapeDtypeStruct((_N, _H), jnp.float32),
            jax.ShapeDtypeStruct((8, _H), jnp.float32),
        ],
    )(base, a0, a1, W, b)


def _bn_scale_shift(st_ref, gamma, beta):
    mean = st_ref[0:1, :] * (1.0 / _N)
    var = st_ref[1:2, :] * (1.0 / _N) - mean * mean
    scale = gamma * lax.rsqrt(var + _EPS)
    shift = beta - mean * scale
    return scale, shift


def _bn_mm(hp, st, gamma, beta, W, b):
    def body(hp_ref, st_ref, g_ref, be_ref, W_ref, b_ref, o_ref, st2_ref):
        i = pl.program_id(0)
        scale, shift = _bn_scale_shift(st_ref, g_ref[...], be_ref[...])
        r = jnp.maximum(hp_ref[...] * scale + shift, 0.0)
        hp2 = jnp.dot(r, W_ref[...], precision=_PREC,
                      preferred_element_type=jnp.float32) + b_ref[...]
        o_ref[...] = hp2

        @pl.when(i == 0)
        def _():
            st2_ref[...] = jnp.zeros_like(st2_ref)

        st2_ref[0:1, :] += jnp.sum(hp2, axis=0, keepdims=True)
        st2_ref[1:2, :] += jnp.sum(hp2 * hp2, axis=0, keepdims=True)

    return pl.pallas_call(
        body,
        grid=(_NB,),
        in_specs=[
            pl.BlockSpec((_R, _H), lambda i: (i, 0)),
            pl.BlockSpec((8, _H), lambda i: (0, 0)),
            pl.BlockSpec((1, _H), lambda i: (0, 0)),
            pl.BlockSpec((1, _H), lambda i: (0, 0)),
            pl.BlockSpec((_H, _H), lambda i: (0, 0)),
            pl.BlockSpec((1, _H), lambda i: (0, 0)),
        ],
        out_specs=[
            pl.BlockSpec((_R, _H), lambda i: (i, 0)),
            pl.BlockSpec((8, _H), lambda i: (0, 0)),
        ],
        out_shape=[
            jax.ShapeDtypeStruct((_N, _H), jnp.float32),
            jax.ShapeDtypeStruct((8, _H), jnp.float32),
        ],
    )(hp, st, gamma, beta, W, b)


def _bn_res(hp, st, gamma, beta, res):
    has_res = res is not None

    def body(*refs):
        if has_res:
            hp_ref, st_ref, g_ref, be_ref, r_ref, o_ref = refs
        else:
            hp_ref, st_ref, g_ref, be_ref, o_ref = refs
        scale, shift = _bn_scale_shift(st_ref, g_ref[...], be_ref[...])
        h = jnp.maximum(hp_ref[...] * scale + shift, 0.0)
        if has_res:
            h = h + r_ref[...]
        o_ref[...] = h

    in_specs = [
        pl.BlockSpec((_R, _H), lambda i: (i, 0)),
        pl.BlockSpec((8, _H), lambda i: (0, 0)),
        pl.BlockSpec((1, _H), lambda i: (0, 0)),
        pl.BlockSpec((1, _H), lambda i: (0, 0)),
    ]
    args = [hp, st, gamma, beta]
    if has_res:
        in_specs.append(pl.BlockSpec((_R, _H), lambda i: (i, 0)))
        args.append(res)

    return pl.pallas_call(
        body,
        grid=(_NB,),
        in_specs=in_specs,
        out_specs=pl.BlockSpec((_R, _H), lambda i: (i, 0)),
        out_shape=jax.ShapeDtypeStruct((_N, _H), jnp.float32),
    )(*args)


def _pool(h, batch_f):
    def body(b_ref, h_ref, g_ref):
        i = pl.program_id(0)
        labels = b_ref[0, 0, :]
        seg = lax.broadcasted_iota(jnp.int32, (_R, _G), 1).astype(jnp.float32)
        onehot = jnp.where(labels[:, None] == seg, 1.0, 0.0)
        part = lax.dot_general(onehot, h_ref[...], (((0,), (0,)), ((), ())),
                               precision=_PREC,
                               preferred_element_type=jnp.float32)

        @pl.when(i == 0)
        def _():
            g_ref[...] = jnp.zeros_like(g_ref)

        g_ref[...] += part

    return pl.pallas_call(
        body,
        grid=(_NB,),
        in_specs=[
            pl.BlockSpec((1, 1, _R), lambda i: (i, 0, 0)),
            pl.BlockSpec((_R, _H), lambda i: (i, 0)),
        ],
        out_specs=pl.BlockSpec((_G, _H), lambda i: (0, 0)),
        out_shape=jax.ShapeDtypeStruct((_G, _H), jnp.float32),
    )(batch_f, h)


def _heads(g, hp):
    names = ['head_color', 'head_size', 'head_ground', 'head_struct']
    douts = [16, 16, 8, 32]

    def body(*refs):
        g_ref = refs[0]
        outs = refs[1 + 4 * len(names):]
        gv = g_ref[...]
        for n in range(len(names)):
            W1, b1, W2, b2 = refs[1 + 4 * n:1 + 4 * (n + 1)]
            r = jnp.maximum(
                jnp.dot(gv, W1[...], precision=_PREC,
                        preferred_element_type=jnp.float32) + b1[...], 0.0)
            z = jnp.dot(r, W2[...], precision=_PREC,
                        preferred_element_type=jnp.float32) + b2[...]
            nrm = jnp.sqrt(jnp.sum(z * z, axis=1, keepdims=True))
            outs[n][...] = z / jnp.maximum(nrm, 1e-12)

    args = [g]
    for n in names:
        p = hp[n]
        args += [p['W1'], p['b1'].reshape(1, -1), p['W2'],
                 p['b2'].reshape(1, -1)]

    return pl.pallas_call(
        body,
        out_shape=[jax.ShapeDtypeStruct((_G, d), jnp.float32) for d in douts],
    )(*args)


def kernel(x, edge_index, batch, params):
    src = edge_index[0]
    dst = edge_index[1]
    npad = _EPW - _E // _NW
    dummy_dst = jnp.broadcast_to(
        _DUMMY + (jnp.arange(npad, dtype=jnp.int32) % (_NP - _N)),
        (_NW, npad))
    pad_src = jnp.concatenate(
        [src.reshape(_NW, _E // _NW),
         jnp.zeros((_NW, npad), jnp.int32)], axis=1).reshape(_NW, _NCH, _CH)
    pad_dst = jnp.concatenate(
        [dst.reshape(_NW, _E // _NW), dummy_dst],
        axis=1).reshape(_NW, _NCH, _CH)
    z128 = jnp.zeros((_RPT, _D), jnp.float32)
    z64 = jnp.zeros((_RPT, _H), jnp.float32)
    batch_f = batch.astype(jnp.float32).reshape(_NB, 1, _R)

    seg128 = _seg_sum_sc(_D)
    seg64 = _seg_sum_sc(_H)

    def gin(hin, zblk, dh, p, res, seg):
        agg = seg(hin, pad_src, pad_dst, zblk)
        a0 = lax.slice(agg, (0, 0, 0), (1, _N, dh)).reshape(_N, dh)
        a1 = lax.slice(agg, (1, 0, 0), (2, _N, dh)).reshape(_N, dh)
        hp, st = _mm_stats(hin, a0, a1, p['W1'], p['b1'].reshape(1, -1), dh)
        hp2, st2 = _bn_mm(hp, st, p['g1'].reshape(1, -1),
                          p['be1'].reshape(1, -1), p['W2'],
                          p['b2'].reshape(1, -1))
        return _bn_res(hp2, st2, p['g2'].reshape(1, -1),
                       p['be2'].reshape(1, -1), res)

    h1 = gin(x, z128, _D, params['conv1'], None, seg128)
    h2 = gin(h1, z64, _H, params['conv2'], h1, seg64)
    h3 = gin(h2, z64, _H, params['conv3'], h2, seg64)
    g = _pool(h3, batch_f)
    return _heads(g, params)


# speedup vs baseline: 6.1816x; 1.0824x over previous
import functools

import jax
import jax.numpy as jnp
from jax import lax
from jax.experimental import pallas as pl
from jax.experimental.pallas import tpu as pltpu
from jax.experimental.pallas import tpu_sc as plsc

_N = 10000
_E = 640000
_D = 128
_H = 64
_G = 64

_NW = 32
_CH = 128
_SLAB = 32
_NSL = 5
_NCH = _SLAB * _NSL
_EPW = _CH * _NCH
_EPAD = _NW * _EPW
_RPT = 640
_NP = 16 * _RPT
_DUMMY = _N

_NB = 10
_R = _N // _NB
_EPS = 1e-5
_PREC = lax.Precision.HIGHEST


def _seg_sum_sc(dh):
    che = 8192 // dh
    nsl = _EPW // (_SLAB * che)
    mesh = plsc.VectorSubcoreMesh(core_axis_name="c", subcore_axis_name="s")

    @functools.partial(
        pl.kernel,
        out_type=jax.ShapeDtypeStruct((2, _NP, dh), jnp.float32),
        mesh=mesh,
        compiler_params=pltpu.CompilerParams(use_tc_tiling_on_sc=False),
        scratch_types=[
            pltpu.VMEM((_SLAB, che), jnp.int32),
            pltpu.VMEM((_SLAB, che), jnp.int32),
            pltpu.VMEM((4, che, dh), jnp.float32),
            pltpu.VMEM_SHARED((_NP, dh), jnp.float32),
            pltpu.SemaphoreType.DMA,
            pltpu.SemaphoreType.DMA,
            pltpu.SemaphoreType.DMA,
            pltpu.SemaphoreType.DMA,
            pltpu.SemaphoreType.DMA,
            pltpu.SemaphoreType.DMA,
            pltpu.SemaphoreType.DMA,
            pltpu.SemaphoreType.DMA,
        ],
    )
    def k(h_hbm, src_hbm, dst_hbm, z_hbm, out_hbm, src_v, dst_v, rows_v,
          agg_sh, g0, g1, g2, g3, s0, s1, s2, s3):
        sg = [g0, g1, g2, g3]
        ss = [s0, s1, s2, s3]
        c = lax.axis_index("c")
        s = lax.axis_index("s")
        wid = s * 2 + c
        pltpu.sync_copy(z_hbm, agg_sh.at[pl.ds(s * _RPT, _RPT)])
        plsc.subcore_barrier()

        def gather(j, kk):
            return pltpu.make_async_copy(
                h_hbm.at[src_v.at[j]], rows_v.at[kk], sg[kk])

        def scatter(j, kk):
            return pltpu.make_async_copy(
                rows_v.at[kk], agg_sh.at[dst_v.at[j]], ss[kk])

        def slab(t, carry):
            pltpu.sync_copy(src_hbm.at[wid, pl.ds(t * _SLAB, _SLAB)], src_v)
            pltpu.sync_copy(dst_hbm.at[wid, pl.ds(t * _SLAB, _SLAB)], dst_v)
            for kk in range(4):
                gather(kk, kk).start()

            def grp(q, carry2):
                for kk in range(4):
                    gather(4 * q + kk, kk).wait()
                    scatter(4 * q + kk, kk).start(add=True)
                for kk in range(4):
                    scatter(4 * q + kk, kk).wait()
                    gather(4 * (q + 1) + kk, kk).start()
                return carry2

            lax.fori_loop(0, _SLAB // 4 - 1, grp, carry)
            last = _SLAB - 4
            for kk in range(4):
                gather(last + kk, kk).wait()
                scatter(last + kk, kk).start(add=True)
            for kk in range(4):
                scatter(last + kk, kk).wait()
            return carry

        lax.fori_loop(0, nsl, slab, 0)
        plsc.subcore_barrier()
        pltpu.sync_copy(agg_sh.at[pl.ds(s * _RPT, _RPT)],
                        out_hbm.at[c, pl.ds(s * _RPT, _RPT)])

    return k, che


def _mm_stats(base, a0, a1, W, b, din):
    def body(base_ref, a0_ref, a1_ref, W_ref, b_ref, hp_ref, st_ref):
        i = pl.program_id(0)
        u = base_ref[...] + a0_ref[...] + a1_ref[...]
        hp = jnp.dot(u, W_ref[...], precision=_PREC,
                     preferred_element_type=jnp.float32) + b_ref[...]
        hp_ref[...] = hp

        @pl.when(i == 0)
        def _():
            st_ref[...] = jnp.zeros_like(st_ref)

        st_ref[0:1, :] += jnp.sum(hp, axis=0, keepdims=True)
        st_ref[1:2, :] += jnp.sum(hp * hp, axis=0, keepdims=True)

    return pl.pallas_call(
        body,
        grid=(_NB,),
        in_specs=[
            pl.BlockSpec((_R, din), lambda i: (i, 0)),
            pl.BlockSpec((_R, din), lambda i: (i, 0)),
            pl.BlockSpec((_R, din), lambda i: (i, 0)),
            pl.BlockSpec((din, _H), lambda i: (0, 0)),
            pl.BlockSpec((1, _H), lambda i: (0, 0)),
        ],
        out_specs=[
            pl.BlockSpec((_R, _H), lambda i: (i, 0)),
            pl.BlockSpec((8, _H), lambda i: (0, 0)),
        ],
        out_shape=[
            jax.ShapeDtypeStruct((_N, _H), jnp.float32),
            jax.ShapeDtypeStruct((8, _H), jnp.float32),
        ],
    )(base, a0, a1, W, b)


def _bn_scale_shift(st_ref, gamma, beta):
    mean = st_ref[0:1, :] * (1.0 / _N)
    var = st_ref[1:2, :] * (1.0 / _N) - mean * mean
    scale = gamma * lax.rsqrt(var + _EPS)
    shift = beta - mean * scale
    return scale, shift


def _bn_mm(hp, st, gamma, beta, W, b):
    def body(hp_ref, st_ref, g_ref, be_ref, W_ref, b_ref, o_ref, st2_ref):
        i = pl.program_id(0)
        scale, shift = _bn_scale_shift(st_ref, g_ref[...], be_ref[...])
        r = jnp.maximum(hp_ref[...] * scale + shift, 0.0)
        hp2 = jnp.dot(r, W_ref[...], precision=_PREC,
                      preferred_element_type=jnp.float32) + b_ref[...]
        o_ref[...] = hp2

        @pl.when(i == 0)
        def _():
            st2_ref[...] = jnp.zeros_like(st2_ref)

        st2_ref[0:1, :] += jnp.sum(hp2, axis=0, keepdims=True)
        st2_ref[1:2, :] += jnp.sum(hp2 * hp2, axis=0, keepdims=True)

    return pl.pallas_call(
        body,
        grid=(_NB,),
        in_specs=[
            pl.BlockSpec((_R, _H), lambda i: (i, 0)),
            pl.BlockSpec((8, _H), lambda i: (0, 0)),
            pl.BlockSpec((1, _H), lambda i: (0, 0)),
            pl.BlockSpec((1, _H), lambda i: (0, 0)),
            pl.BlockSpec((_H, _H), lambda i: (0, 0)),
            pl.BlockSpec((1, _H), lambda i: (0, 0)),
        ],
        out_specs=[
            pl.BlockSpec((_R, _H), lambda i: (i, 0)),
            pl.BlockSpec((8, _H), lambda i: (0, 0)),
        ],
        out_shape=[
            jax.ShapeDtypeStruct((_N, _H), jnp.float32),
            jax.ShapeDtypeStruct((8, _H), jnp.float32),
        ],
    )(hp, st, gamma, beta, W, b)


def _bn_res(hp, st, gamma, beta, res):
    has_res = res is not None

    def body(*refs):
        if has_res:
            hp_ref, st_ref, g_ref, be_ref, r_ref, o_ref = refs
        else:
            hp_ref, st_ref, g_ref, be_ref, o_ref = refs
        scale, shift = _bn_scale_shift(st_ref, g_ref[...], be_ref[...])
        h = jnp.maximum(hp_ref[...] * scale + shift, 0.0)
        if has_res:
            h = h + r_ref[...]
        o_ref[...] = h

    in_specs = [
        pl.BlockSpec((_R, _H), lambda i: (i, 0)),
        pl.BlockSpec((8, _H), lambda i: (0, 0)),
        pl.BlockSpec((1, _H), lambda i: (0, 0)),
        pl.BlockSpec((1, _H), lambda i: (0, 0)),
    ]
    args = [hp, st, gamma, beta]
    if has_res:
        in_specs.append(pl.BlockSpec((_R, _H), lambda i: (i, 0)))
        args.append(res)

    return pl.pallas_call(
        body,
        grid=(_NB,),
        in_specs=in_specs,
        out_specs=pl.BlockSpec((_R, _H), lambda i: (i, 0)),
        out_shape=jax.ShapeDtypeStruct((_N, _H), jnp.float32),
    )(*args)


def _pool(h, batch_f):
    def body(b_ref, h_ref, g_ref):
        i = pl.program_id(0)
        labels = b_ref[0, 0, :]
        seg = lax.broadcasted_iota(jnp.int32, (_R, _G), 1).astype(jnp.float32)
        onehot = jnp.where(labels[:, None] == seg, 1.0, 0.0)
        part = lax.dot_general(onehot, h_ref[...], (((0,), (0,)), ((), ())),
                               precision=_PREC,
                               preferred_element_type=jnp.float32)

        @pl.when(i == 0)
        def _():
            g_ref[...] = jnp.zeros_like(g_ref)

        g_ref[...] += part

    return pl.pallas_call(
        body,
        grid=(_NB,),
        in_specs=[
            pl.BlockSpec((1, 1, _R), lambda i: (i, 0, 0)),
            pl.BlockSpec((_R, _H), lambda i: (i, 0)),
        ],
        out_specs=pl.BlockSpec((_G, _H), lambda i: (0, 0)),
        out_shape=jax.ShapeDtypeStruct((_G, _H), jnp.float32),
    )(batch_f, h)


def _heads(g, hp):
    names = ['head_color', 'head_size', 'head_ground', 'head_struct']
    douts = [16, 16, 8, 32]

    def body(*refs):
        g_ref = refs[0]
        outs = refs[1 + 4 * len(names):]
        gv = g_ref[...]
        for n in range(len(names)):
            W1, b1, W2, b2 = refs[1 + 4 * n:1 + 4 * (n + 1)]
            r = jnp.maximum(
                jnp.dot(gv, W1[...], precision=_PREC,
                        preferred_element_type=jnp.float32) + b1[...], 0.0)
            z = jnp.dot(r, W2[...], precision=_PREC,
                        preferred_element_type=jnp.float32) + b2[...]
            nrm = jnp.sqrt(jnp.sum(z * z, axis=1, keepdims=True))
            outs[n][...] = z / jnp.maximum(nrm, 1e-12)

    args = [g]
    for n in names:
        p = hp[n]
        args += [p['W1'], p['b1'].reshape(1, -1), p['W2'],
                 p['b2'].reshape(1, -1)]

    return pl.pallas_call(
        body,
        out_shape=[jax.ShapeDtypeStruct((_G, d), jnp.float32) for d in douts],
    )(*args)


def kernel(x, edge_index, batch, params):
    src = edge_index[0]
    dst = edge_index[1]
    npad = _EPW - _E // _NW
    dummy_dst = jnp.broadcast_to(
        _DUMMY + (jnp.arange(npad, dtype=jnp.int32) % (_NP - _N)),
        (_NW, npad))
    pad_src = jnp.concatenate(
        [src.reshape(_NW, _E // _NW),
         jnp.zeros((_NW, npad), jnp.int32)], axis=1)
    pad_dst = jnp.concatenate(
        [dst.reshape(_NW, _E // _NW), dummy_dst], axis=1)
    z128 = jnp.zeros((_RPT, _D), jnp.float32)
    z64 = jnp.zeros((_RPT, _H), jnp.float32)
    batch_f = batch.astype(jnp.float32).reshape(_NB, 1, _R)

    seg128, che128 = _seg_sum_sc(_D)
    seg64, che64 = _seg_sum_sc(_H)

    def gin(hin, zblk, dh, p, res, seg, che):
        agg = seg(hin, pad_src.reshape(_NW, _EPW // che, che),
                  pad_dst.reshape(_NW, _EPW // che, che), zblk)
        a0 = lax.slice(agg, (0, 0, 0), (1, _N, dh)).reshape(_N, dh)
        a1 = lax.slice(agg, (1, 0, 0), (2, _N, dh)).reshape(_N, dh)
        hp, st = _mm_stats(hin, a0, a1, p['W1'], p['b1'].reshape(1, -1), dh)
        hp2, st2 = _bn_mm(hp, st, p['g1'].reshape(1, -1),
                          p['be1'].reshape(1, -1), p['W2'],
                          p['b2'].reshape(1, -1))
        return _bn_res(hp2, st2, p['g2'].reshape(1, -1),
                       p['be2'].reshape(1, -1), res)

    h1 = gin(x, z128, _D, params['conv1'], None, seg128, che128)
    h2 = gin(h1, z64, _H, params['conv2'], h1, seg64, che64)
    h3 = gin(h2, z64, _H, params['conv3'], h2, seg64, che64)
    g = _pool(h3, batch_f)
    return _heads(g, params)


# speedup vs baseline: 7.6645x; 1.2399x over previous
import functools

import jax
import jax.numpy as jnp
from jax import lax
from jax.experimental import pallas as pl
from jax.experimental.pallas import tpu as pltpu
from jax.experimental.pallas import tpu_sc as plsc

_N = 10000
_E = 640000
_D = 128
_H = 64
_G = 64

_NW = 32
_CH = 128
_SLAB = 32
_NSL = 5
_NCH = _SLAB * _NSL
_EPW = _CH * _NCH
_EPAD = _NW * _EPW
_RPT = 640
_NP = 16 * _RPT
_DUMMY = _N

_NB = 10
_R = _N // _NB
_EPS = 1e-5
_PREC = lax.Precision.HIGHEST


def _seg_sum_sc(dh):
    che = 8192 // dh
    nsl = _EPW // (_SLAB * che)
    mesh = plsc.VectorSubcoreMesh(core_axis_name="c", subcore_axis_name="s")

    @functools.partial(
        pl.kernel,
        out_type=jax.ShapeDtypeStruct((2, _NP, dh), jnp.float32),
        mesh=mesh,
        compiler_params=pltpu.CompilerParams(use_tc_tiling_on_sc=False),
        scratch_types=[
            pltpu.VMEM((_SLAB, che), jnp.int32),
            pltpu.VMEM((_SLAB, che), jnp.int32),
            pltpu.VMEM((4, che, dh), jnp.float32),
            pltpu.VMEM_SHARED((_NP, dh), jnp.float32),
            pltpu.SemaphoreType.DMA,
            pltpu.SemaphoreType.DMA,
            pltpu.SemaphoreType.DMA,
            pltpu.SemaphoreType.DMA,
            pltpu.SemaphoreType.DMA,
            pltpu.SemaphoreType.DMA,
            pltpu.SemaphoreType.DMA,
            pltpu.SemaphoreType.DMA,
        ],
    )
    def k(h_hbm, src_hbm, dst_hbm, z_hbm, out_hbm, src_v, dst_v, rows_v,
          agg_sh, g0, g1, g2, g3, s0, s1, s2, s3):
        sg = [g0, g1, g2, g3]
        ss = [s0, s1, s2, s3]
        c = lax.axis_index("c")
        s = lax.axis_index("s")
        wid = s * 2 + c
        pltpu.sync_copy(z_hbm, agg_sh.at[pl.ds(s * _RPT, _RPT)])
        plsc.subcore_barrier()

        def gather(j, kk):
            return pltpu.make_async_copy(
                h_hbm.at[src_v.at[j]], rows_v.at[kk], sg[kk])

        def scatter(j, kk):
            return pltpu.make_async_copy(
                rows_v.at[kk], agg_sh.at[dst_v.at[j]], ss[kk])

        def slab(t, carry):
            pltpu.sync_copy(src_hbm.at[wid, pl.ds(t * _SLAB, _SLAB)], src_v)
            pltpu.sync_copy(dst_hbm.at[wid, pl.ds(t * _SLAB, _SLAB)], dst_v)
            for kk in range(4):
                gather(kk, kk).start()

            def grp(q, carry2):
                for kk in range(4):
                    gather(4 * q + kk, kk).wait()
                    scatter(4 * q + kk, kk).start(add=True)
                for kk in range(4):
                    scatter(4 * q + kk, kk).wait()
                    gather(4 * (q + 1) + kk, kk).start()
                return carry2

            lax.fori_loop(0, _SLAB // 4 - 1, grp, carry)
            last = _SLAB - 4
            for kk in range(4):
                gather(last + kk, kk).wait()
                scatter(last + kk, kk).start(add=True)
            for kk in range(4):
                scatter(last + kk, kk).wait()
            return carry

        lax.fori_loop(0, nsl, slab, 0)
        plsc.subcore_barrier()
        pltpu.sync_copy(agg_sh.at[pl.ds(s * _RPT, _RPT)],
                        out_hbm.at[c, pl.ds(s * _RPT, _RPT)])

    return k, che


def _pre_mm(hin, W, din):
    def body(h_ref, W_ref, y_ref):
        y_ref[...] = jnp.dot(h_ref[...], W_ref[...], precision=_PREC,
                             preferred_element_type=jnp.float32)

    return pl.pallas_call(
        body,
        grid=(_NB,),
        in_specs=[
            pl.BlockSpec((_R, din), lambda i: (i, 0)),
            pl.BlockSpec((din, _H), lambda i: (0, 0)),
        ],
        out_specs=pl.BlockSpec((_R, _H), lambda i: (i, 0)),
        out_shape=jax.ShapeDtypeStruct((_N, _H), jnp.float32),
    )(hin, W)


def _add_stats(y, a0, a1, b):
    def body(y_ref, a0_ref, a1_ref, b_ref, hp_ref, st_ref):
        i = pl.program_id(0)
        hp = y_ref[...] + a0_ref[...] + a1_ref[...] + b_ref[...]
        hp_ref[...] = hp

        @pl.when(i == 0)
        def _():
            st_ref[...] = jnp.zeros_like(st_ref)

        st_ref[0:1, :] += jnp.sum(hp, axis=0, keepdims=True)
        st_ref[1:2, :] += jnp.sum(hp * hp, axis=0, keepdims=True)

    return pl.pallas_call(
        body,
        grid=(_NB,),
        in_specs=[
            pl.BlockSpec((_R, _H), lambda i: (i, 0)),
            pl.BlockSpec((_R, _H), lambda i: (i, 0)),
            pl.BlockSpec((_R, _H), lambda i: (i, 0)),
            pl.BlockSpec((1, _H), lambda i: (0, 0)),
        ],
        out_specs=[
            pl.BlockSpec((_R, _H), lambda i: (i, 0)),
            pl.BlockSpec((8, _H), lambda i: (0, 0)),
        ],
        out_shape=[
            jax.ShapeDtypeStruct((_N, _H), jnp.float32),
            jax.ShapeDtypeStruct((8, _H), jnp.float32),
        ],
    )(y, a0, a1, b)


def _bn_scale_shift(st_ref, gamma, beta):
    mean = st_ref[0:1, :] * (1.0 / _N)
    var = st_ref[1:2, :] * (1.0 / _N) - mean * mean
    scale = gamma * lax.rsqrt(var + _EPS)
    shift = beta - mean * scale
    return scale, shift


def _bn_mm(hp, st, gamma, beta, W, b):
    def body(hp_ref, st_ref, g_ref, be_ref, W_ref, b_ref, o_ref, st2_ref):
        i = pl.program_id(0)
        scale, shift = _bn_scale_shift(st_ref, g_ref[...], be_ref[...])
        r = jnp.maximum(hp_ref[...] * scale + shift, 0.0)
        hp2 = jnp.dot(r, W_ref[...], precision=_PREC,
                      preferred_element_type=jnp.float32) + b_ref[...]
        o_ref[...] = hp2

        @pl.when(i == 0)
        def _():
            st2_ref[...] = jnp.zeros_like(st2_ref)

        st2_ref[0:1, :] += jnp.sum(hp2, axis=0, keepdims=True)
        st2_ref[1:2, :] += jnp.sum(hp2 * hp2, axis=0, keepdims=True)

    return pl.pallas_call(
        body,
        grid=(_NB,),
        in_specs=[
            pl.BlockSpec((_R, _H), lambda i: (i, 0)),
            pl.BlockSpec((8, _H), lambda i: (0, 0)),
            pl.BlockSpec((1, _H), lambda i: (0, 0)),
            pl.BlockSpec((1, _H), lambda i: (0, 0)),
            pl.BlockSpec((_H, _H), lambda i: (0, 0)),
            pl.BlockSpec((1, _H), lambda i: (0, 0)),
        ],
        out_specs=[
            pl.BlockSpec((_R, _H), lambda i: (i, 0)),
            pl.BlockSpec((8, _H), lambda i: (0, 0)),
        ],
        out_shape=[
            jax.ShapeDtypeStruct((_N, _H), jnp.float32),
            jax.ShapeDtypeStruct((8, _H), jnp.float32),
        ],
    )(hp, st, gamma, beta, W, b)


def _bn_res(hp, st, gamma, beta, res):
    has_res = res is not None

    def body(*refs):
        if has_res:
            hp_ref, st_ref, g_ref, be_ref, r_ref, o_ref = refs
        else:
            hp_ref, st_ref, g_ref, be_ref, o_ref = refs
        scale, shift = _bn_scale_shift(st_ref, g_ref[...], be_ref[...])
        h = jnp.maximum(hp_ref[...] * scale + shift, 0.0)
        if has_res:
            h = h + r_ref[...]
        o_ref[...] = h

    in_specs = [
        pl.BlockSpec((_R, _H), lambda i: (i, 0)),
        pl.BlockSpec((8, _H), lambda i: (0, 0)),
        pl.BlockSpec((1, _H), lambda i: (0, 0)),
        pl.BlockSpec((1, _H), lambda i: (0, 0)),
    ]
    args = [hp, st, gamma, beta]
    if has_res:
        in_specs.append(pl.BlockSpec((_R, _H), lambda i: (i, 0)))
        args.append(res)

    return pl.pallas_call(
        body,
        grid=(_NB,),
        in_specs=in_specs,
        out_specs=pl.BlockSpec((_R, _H), lambda i: (i, 0)),
        out_shape=jax.ShapeDtypeStruct((_N, _H), jnp.float32),
    )(*args)


def _pool(h, batch_f):
    def body(b_ref, h_ref, g_ref):
        i = pl.program_id(0)
        labels = b_ref[0, 0, :]
        seg = lax.broadcasted_iota(jnp.int32, (_R, _G), 1).astype(jnp.float32)
        onehot = jnp.where(labels[:, None] == seg, 1.0, 0.0)
        part = lax.dot_general(onehot, h_ref[...], (((0,), (0,)), ((), ())),
                               precision=_PREC,
                               preferred_element_type=jnp.float32)

        @pl.when(i == 0)
        def _():
            g_ref[...] = jnp.zeros_like(g_ref)

        g_ref[...] += part

    return pl.pallas_call(
        body,
        grid=(_NB,),
        in_specs=[
            pl.BlockSpec((1, 1, _R), lambda i: (i, 0, 0)),
            pl.BlockSpec((_R, _H), lambda i: (i, 0)),
        ],
        out_specs=pl.BlockSpec((_G, _H), lambda i: (0, 0)),
        out_shape=jax.ShapeDtypeStruct((_G, _H), jnp.float32),
    )(batch_f, h)


def _heads(g, hp):
    names = ['head_color', 'head_size', 'head_ground', 'head_struct']
    douts = [16, 16, 8, 32]

    def body(*refs):
        g_ref = refs[0]
        outs = refs[1 + 4 * len(names):]
        gv = g_ref[...]
        for n in range(len(names)):
            W1, b1, W2, b2 = refs[1 + 4 * n:1 + 4 * (n + 1)]
            r = jnp.maximum(
                jnp.dot(gv, W1[...], precision=_PREC,
                        preferred_element_type=jnp.float32) + b1[...], 0.0)
            z = jnp.dot(r, W2[...], precision=_PREC,
                        preferred_element_type=jnp.float32) + b2[...]
            nrm = jnp.sqrt(jnp.sum(z * z, axis=1, keepdims=True))
            outs[n][...] = z / jnp.maximum(nrm, 1e-12)

    args = [g]
    for n in names:
        p = hp[n]
        args += [p['W1'], p['b1'].reshape(1, -1), p['W2'],
                 p['b2'].reshape(1, -1)]

    return pl.pallas_call(
        body,
        out_shape=[jax.ShapeDtypeStruct((_G, d), jnp.float32) for d in douts],
    )(*args)


def kernel(x, edge_index, batch, params):
    src = edge_index[0]
    dst = edge_index[1]
    npad = _EPW - _E // _NW
    dummy_dst = jnp.broadcast_to(
        _DUMMY + (jnp.arange(npad, dtype=jnp.int32) % (_NP - _N)),
        (_NW, npad))
    pad_src = jnp.concatenate(
        [src.reshape(_NW, _E // _NW),
         jnp.zeros((_NW, npad), jnp.int32)], axis=1)
    pad_dst = jnp.concatenate(
        [dst.reshape(_NW, _E // _NW), dummy_dst], axis=1)
    zblk = jnp.zeros((_RPT, _H), jnp.float32)
    batch_f = batch.astype(jnp.float32).reshape(_NB, 1, _R)

    seg, che = _seg_sum_sc(_H)
    srcp = pad_src.reshape(_NW, _EPW // che, che)
    dstp = pad_dst.reshape(_NW, _EPW // che, che)

    def gin(hin, din, p, res):
        y = _pre_mm(hin, p['W1'], din)
        agg = seg(y, srcp, dstp, zblk)
        a0 = lax.slice(agg, (0, 0, 0), (1, _N, _H)).reshape(_N, _H)
        a1 = lax.slice(agg, (1, 0, 0), (2, _N, _H)).reshape(_N, _H)
        hp, st = _add_stats(y, a0, a1, p['b1'].reshape(1, -1))
        hp2, st2 = _bn_mm(hp, st, p['g1'].reshape(1, -1),
                          p['be1'].reshape(1, -1), p['W2'],
                          p['b2'].reshape(1, -1))
        return _bn_res(hp2, st2, p['g2'].reshape(1, -1),
                       p['be2'].reshape(1, -1), res)

    h1 = gin(x, _D, params['conv1'], None)
    h2 = gin(h1, _H, params['conv2'], h1)
    h3 = gin(h2, _H, params['conv3'], h2)
    g = _pool(h3, batch_f)
    return _heads(g, params)


# speedup vs baseline: 14.0087x; 1.8277x over previous
import functools

import jax
import jax.numpy as jnp
from jax import lax
from jax.experimental import pallas as pl
from jax.experimental.pallas import tpu as pltpu
from jax.experimental.pallas import tpu_sc as plsc

_N = 10000
_E = 640000
_D = 128
_H = 64
_G = 64

_NW = 32
_CH = 128
_SLAB = 32
_NSL = 5
_NCH = _SLAB * _NSL
_EPW = _CH * _NCH
_EPAD = _NW * _EPW
_RPT = 640
_NP = 16 * _RPT
_DUMMY = _N

_NB = 10
_R = _N // _NB
_EPS = 1e-5
_PREC = lax.Precision.HIGHEST


def _seg_sum_sc(dh):
    che = 8192 // dh
    nsl = _EPW // (_SLAB * che)
    mesh = plsc.VectorSubcoreMesh(core_axis_name="c", subcore_axis_name="s")

    @functools.partial(
        pl.kernel,
        out_type=jax.ShapeDtypeStruct((2, _NP, dh), jnp.float32),
        mesh=mesh,
        compiler_params=pltpu.CompilerParams(use_tc_tiling_on_sc=False),
        scratch_types=[
            pltpu.VMEM((_SLAB, che), jnp.int32),
            pltpu.VMEM((_SLAB, che), jnp.int32),
            pltpu.VMEM((4, che, dh), jnp.float32),
            pltpu.VMEM_SHARED((_NP, dh), jnp.float32),
            pltpu.VMEM_SHARED((_N, dh), jnp.float32),
            pltpu.SemaphoreType.DMA,
            pltpu.SemaphoreType.DMA,
            pltpu.SemaphoreType.DMA,
            pltpu.SemaphoreType.DMA,
            pltpu.SemaphoreType.DMA,
            pltpu.SemaphoreType.DMA,
            pltpu.SemaphoreType.DMA,
            pltpu.SemaphoreType.DMA,
        ],
    )
    def k(h_hbm, src_hbm, dst_hbm, z_hbm, out_hbm, src_v, dst_v, rows_v,
          agg_sh, tab_sh, g0, g1, g2, g3, s0, s1, s2, s3):
        sg = [g0, g1, g2, g3]
        ss = [s0, s1, s2, s3]
        c = lax.axis_index("c")
        s = lax.axis_index("s")
        wid = s * 2 + c
        pltpu.sync_copy(z_hbm, agg_sh.at[pl.ds(s * _RPT, _RPT)])
        pltpu.sync_copy(h_hbm.at[pl.ds(s * (_N // 16), _N // 16)],
                        tab_sh.at[pl.ds(s * (_N // 16), _N // 16)])
        plsc.subcore_barrier()

        def gather(j, kk):
            return pltpu.make_async_copy(
                tab_sh.at[src_v.at[j]], rows_v.at[kk], sg[kk])

        def scatter(j, kk):
            return pltpu.make_async_copy(
                rows_v.at[kk], agg_sh.at[dst_v.at[j]], ss[kk])

        def slab(t, carry):
            pltpu.sync_copy(src_hbm.at[wid, pl.ds(t * _SLAB, _SLAB)], src_v)
            pltpu.sync_copy(dst_hbm.at[wid, pl.ds(t * _SLAB, _SLAB)], dst_v)
            for kk in range(4):
                gather(kk, kk).start()

            def grp(q, carry2):
                for kk in range(4):
                    gather(4 * q + kk, kk).wait()
                    scatter(4 * q + kk, kk).start(add=True)
                for kk in range(4):
                    scatter(4 * q + kk, kk).wait()
                    gather(4 * (q + 1) + kk, kk).start()
                return carry2

            lax.fori_loop(0, _SLAB // 4 - 1, grp, carry)
            last = _SLAB - 4
            for kk in range(4):
                gather(last + kk, kk).wait()
                scatter(last + kk, kk).start(add=True)
            for kk in range(4):
                scatter(last + kk, kk).wait()
            return carry

        lax.fori_loop(0, nsl, slab, 0)
        plsc.subcore_barrier()
        pltpu.sync_copy(agg_sh.at[pl.ds(s * _RPT, _RPT)],
                        out_hbm.at[c, pl.ds(s * _RPT, _RPT)])

    return k, che


def _pre_mm(hin, W, din):
    def body(h_ref, W_ref, y_ref):
        y_ref[...] = jnp.dot(h_ref[...], W_ref[...], precision=_PREC,
                             preferred_element_type=jnp.float32)

    return pl.pallas_call(
        body,
        grid=(_NB,),
        in_specs=[
            pl.BlockSpec((_R, din), lambda i: (i, 0)),
            pl.BlockSpec((din, _H), lambda i: (0, 0)),
        ],
        out_specs=pl.BlockSpec((_R, _H), lambda i: (i, 0)),
        out_shape=jax.ShapeDtypeStruct((_N, _H), jnp.float32),
    )(hin, W)


def _add_stats(y, a0, a1, b):
    def body(y_ref, a0_ref, a1_ref, b_ref, hp_ref, st_ref):
        i = pl.program_id(0)
        hp = y_ref[...] + a0_ref[...] + a1_ref[...] + b_ref[...]
        hp_ref[...] = hp

        @pl.when(i == 0)
        def _():
            st_ref[...] = jnp.zeros_like(st_ref)

        st_ref[0:1, :] += jnp.sum(hp, axis=0, keepdims=True)
        st_ref[1:2, :] += jnp.sum(hp * hp, axis=0, keepdims=True)

    return pl.pallas_call(
        body,
        grid=(_NB,),
        in_specs=[
            pl.BlockSpec((_R, _H), lambda i: (i, 0)),
            pl.BlockSpec((_R, _H), lambda i: (i, 0)),
            pl.BlockSpec((_R, _H), lambda i: (i, 0)),
            pl.BlockSpec((1, _H), lambda i: (0, 0)),
        ],
        out_specs=[
            pl.BlockSpec((_R, _H), lambda i: (i, 0)),
            pl.BlockSpec((8, _H), lambda i: (0, 0)),
        ],
        out_shape=[
            jax.ShapeDtypeStruct((_N, _H), jnp.float32),
            jax.ShapeDtypeStruct((8, _H), jnp.float32),
        ],
    )(y, a0, a1, b)


def _bn_scale_shift(st_ref, gamma, beta):
    mean = st_ref[0:1, :] * (1.0 / _N)
    var = st_ref[1:2, :] * (1.0 / _N) - mean * mean
    scale = gamma * lax.rsqrt(var + _EPS)
    shift = beta - mean * scale
    return scale, shift


def _bn_mm(hp, st, gamma, beta, W, b):
    def body(hp_ref, st_ref, g_ref, be_ref, W_ref, b_ref, o_ref, st2_ref):
        i = pl.program_id(0)
        scale, shift = _bn_scale_shift(st_ref, g_ref[...], be_ref[...])
        r = jnp.maximum(hp_ref[...] * scale + shift, 0.0)
        hp2 = jnp.dot(r, W_ref[...], precision=_PREC,
                      preferred_element_type=jnp.float32) + b_ref[...]
        o_ref[...] = hp2

        @pl.when(i == 0)
        def _():
            st2_ref[...] = jnp.zeros_like(st2_ref)

        st2_ref[0:1, :] += jnp.sum(hp2, axis=0, keepdims=True)
        st2_ref[1:2, :] += jnp.sum(hp2 * hp2, axis=0, keepdims=True)

    return pl.pallas_call(
        body,
        grid=(_NB,),
        in_specs=[
            pl.BlockSpec((_R, _H), lambda i: (i, 0)),
            pl.BlockSpec((8, _H), lambda i: (0, 0)),
            pl.BlockSpec((1, _H), lambda i: (0, 0)),
            pl.BlockSpec((1, _H), lambda i: (0, 0)),
            pl.BlockSpec((_H, _H), lambda i: (0, 0)),
            pl.BlockSpec((1, _H), lambda i: (0, 0)),
        ],
        out_specs=[
            pl.BlockSpec((_R, _H), lambda i: (i, 0)),
            pl.BlockSpec((8, _H), lambda i: (0, 0)),
        ],
        out_shape=[
            jax.ShapeDtypeStruct((_N, _H), jnp.float32),
            jax.ShapeDtypeStruct((8, _H), jnp.float32),
        ],
    )(hp, st, gamma, beta, W, b)


def _bn_res(hp, st, gamma, beta, res):
    has_res = res is not None

    def body(*refs):
        if has_res:
            hp_ref, st_ref, g_ref, be_ref, r_ref, o_ref = refs
        else:
            hp_ref, st_ref, g_ref, be_ref, o_ref = refs
        scale, shift = _bn_scale_shift(st_ref, g_ref[...], be_ref[...])
        h = jnp.maximum(hp_ref[...] * scale + shift, 0.0)
        if has_res:
            h = h + r_ref[...]
        o_ref[...] = h

    in_specs = [
        pl.BlockSpec((_R, _H), lambda i: (i, 0)),
        pl.BlockSpec((8, _H), lambda i: (0, 0)),
        pl.BlockSpec((1, _H), lambda i: (0, 0)),
        pl.BlockSpec((1, _H), lambda i: (0, 0)),
    ]
    args = [hp, st, gamma, beta]
    if has_res:
        in_specs.append(pl.BlockSpec((_R, _H), lambda i: (i, 0)))
        args.append(res)

    return pl.pallas_call(
        body,
        grid=(_NB,),
        in_specs=in_specs,
        out_specs=pl.BlockSpec((_R, _H), lambda i: (i, 0)),
        out_shape=jax.ShapeDtypeStruct((_N, _H), jnp.float32),
    )(*args)


def _pool(h, batch_f):
    def body(b_ref, h_ref, g_ref):
        i = pl.program_id(0)
        labels = b_ref[0, 0, :]
        seg = lax.broadcasted_iota(jnp.int32, (_R, _G), 1).astype(jnp.float32)
        onehot = jnp.where(labels[:, None] == seg, 1.0, 0.0)
        part = lax.dot_general(onehot, h_ref[...], (((0,), (0,)), ((), ())),
                               precision=_PREC,
                               preferred_element_type=jnp.float32)

        @pl.when(i == 0)
        def _():
            g_ref[...] = jnp.zeros_like(g_ref)

        g_ref[...] += part

    return pl.pallas_call(
        body,
        grid=(_NB,),
        in_specs=[
            pl.BlockSpec((1, 1, _R), lambda i: (i, 0, 0)),
            pl.BlockSpec((_R, _H), lambda i: (i, 0)),
        ],
        out_specs=pl.BlockSpec((_G, _H), lambda i: (0, 0)),
        out_shape=jax.ShapeDtypeStruct((_G, _H), jnp.float32),
    )(batch_f, h)


def _heads(g, hp):
    names = ['head_color', 'head_size', 'head_ground', 'head_struct']
    douts = [16, 16, 8, 32]

    def body(*refs):
        g_ref = refs[0]
        outs = refs[1 + 4 * len(names):]
        gv = g_ref[...]
        for n in range(len(names)):
            W1, b1, W2, b2 = refs[1 + 4 * n:1 + 4 * (n + 1)]
            r = jnp.maximum(
                jnp.dot(gv, W1[...], precision=_PREC,
                        preferred_element_type=jnp.float32) + b1[...], 0.0)
            z = jnp.dot(r, W2[...], precision=_PREC,
                        preferred_element_type=jnp.float32) + b2[...]
            nrm = jnp.sqrt(jnp.sum(z * z, axis=1, keepdims=True))
            outs[n][...] = z / jnp.maximum(nrm, 1e-12)

    args = [g]
    for n in names:
        p = hp[n]
        args += [p['W1'], p['b1'].reshape(1, -1), p['W2'],
                 p['b2'].reshape(1, -1)]

    return pl.pallas_call(
        body,
        out_shape=[jax.ShapeDtypeStruct((_G, d), jnp.float32) for d in douts],
    )(*args)


def kernel(x, edge_index, batch, params):
    src = edge_index[0]
    dst = edge_index[1]
    npad = _EPW - _E // _NW
    dummy_dst = jnp.broadcast_to(
        _DUMMY + (jnp.arange(npad, dtype=jnp.int32) % (_NP - _N)),
        (_NW, npad))
    pad_src = jnp.concatenate(
        [src.reshape(_NW, _E // _NW),
         jnp.zeros((_NW, npad), jnp.int32)], axis=1)
    pad_dst = jnp.concatenate(
        [dst.reshape(_NW, _E // _NW), dummy_dst], axis=1)
    zblk = jnp.zeros((_RPT, _H), jnp.float32)
    batch_f = batch.astype(jnp.float32).reshape(_NB, 1, _R)

    seg, che = _seg_sum_sc(_H)
    srcp = pad_src.reshape(_NW, _EPW // che, che)
    dstp = pad_dst.reshape(_NW, _EPW // che, che)

    def gin(hin, din, p, res):
        y = _pre_mm(hin, p['W1'], din)
        agg = seg(y, srcp, dstp, zblk)
        a0 = lax.slice(agg, (0, 0, 0), (1, _N, _H)).reshape(_N, _H)
        a1 = lax.slice(agg, (1, 0, 0), (2, _N, _H)).reshape(_N, _H)
        hp, st = _add_stats(y, a0, a1, p['b1'].reshape(1, -1))
        hp2, st2 = _bn_mm(hp, st, p['g1'].reshape(1, -1),
                          p['be1'].reshape(1, -1), p['W2'],
                          p['b2'].reshape(1, -1))
        return _bn_res(hp2, st2, p['g2'].reshape(1, -1),
                       p['be2'].reshape(1, -1), res)

    h1 = gin(x, _D, params['conv1'], None)
    h2 = gin(h1, _H, params['conv2'], h1)
    h3 = gin(h2, _H, params['conv3'], h2)
    g = _pool(h3, batch_f)
    return _heads(g, params)


# speedup vs baseline: 15.2837x; 1.0910x over previous
import functools

import jax
import jax.numpy as jnp
from jax import lax
from jax.experimental import pallas as pl
from jax.experimental.pallas import tpu as pltpu
from jax.experimental.pallas import tpu_sc as plsc

_N = 10000
_E = 640000
_D = 128
_H = 64
_G = 64

_NW = 32
_CH = 128
_SLAB = 32
_NSL = 5
_NCH = _SLAB * _NSL
_EPW = _CH * _NCH
_EPAD = _NW * _EPW
_RPT = 640
_NP = 16 * _RPT
_DUMMY = _N

_NB = 10
_R = _N // _NB
_EPS = 1e-5
_PREC = lax.Precision.HIGHEST


def _seg_sum_sc(dh):
    che = 8192 // dh
    nsl = _EPW // (_SLAB * che)
    mesh = plsc.VectorSubcoreMesh(core_axis_name="c", subcore_axis_name="s")

    @functools.partial(
        pl.kernel,
        out_type=jax.ShapeDtypeStruct((2, _NP, dh), jnp.float32),
        mesh=mesh,
        compiler_params=pltpu.CompilerParams(use_tc_tiling_on_sc=False),
        scratch_types=[
            pltpu.VMEM((_SLAB, che), jnp.int32),
            pltpu.VMEM((_SLAB, che), jnp.int32),
            pltpu.VMEM((4, che, dh), jnp.float32),
            pltpu.VMEM_SHARED((_NP, dh), jnp.float32),
            pltpu.VMEM_SHARED((_N, dh), jnp.float32),
            pltpu.SemaphoreType.DMA,
            pltpu.SemaphoreType.DMA,
            pltpu.SemaphoreType.DMA,
            pltpu.SemaphoreType.DMA,
            pltpu.SemaphoreType.DMA,
            pltpu.SemaphoreType.DMA,
            pltpu.SemaphoreType.DMA,
            pltpu.SemaphoreType.DMA,
        ],
    )
    def k(h_hbm, src_hbm, dst_hbm, z_hbm, out_hbm, src_v, dst_v, rows_v,
          agg_sh, tab_sh, g0, g1, g2, g3, s0, s1, s2, s3):
        sg = [g0, g1, g2, g3]
        ss = [s0, s1, s2, s3]
        c = lax.axis_index("c")
        s = lax.axis_index("s")
        wid = s * 2 + c
        pltpu.sync_copy(z_hbm, agg_sh.at[pl.ds(s * _RPT, _RPT)])
        pltpu.sync_copy(h_hbm.at[pl.ds(s * (_N // 16), _N // 16)],
                        tab_sh.at[pl.ds(s * (_N // 16), _N // 16)])
        plsc.subcore_barrier()

        def gather(j, kk):
            return pltpu.make_async_copy(
                tab_sh.at[src_v.at[j]], rows_v.at[kk], sg[kk])

        def scatter(j, kk):
            return pltpu.make_async_copy(
                rows_v.at[kk], agg_sh.at[dst_v.at[j]], ss[kk])

        def slab(t, carry):
            pltpu.sync_copy(src_hbm.at[wid, pl.ds(t * _SLAB, _SLAB)], src_v)
            pltpu.sync_copy(dst_hbm.at[wid, pl.ds(t * _SLAB, _SLAB)], dst_v)
            for kk in range(4):
                gather(kk, kk).start()

            def grp(q, carry2):
                for kk in range(4):
                    gather(4 * q + kk, kk).wait()
                    scatter(4 * q + kk, kk).start(add=True)
                for kk in range(4):
                    scatter(4 * q + kk, kk).wait()
                    gather(4 * (q + 1) + kk, kk).start()
                return carry2

            lax.fori_loop(0, _SLAB // 4 - 1, grp, carry)
            last = _SLAB - 4
            for kk in range(4):
                gather(last + kk, kk).wait()
                scatter(last + kk, kk).start(add=True)
            for kk in range(4):
                scatter(last + kk, kk).wait()
            return carry

        lax.fori_loop(0, nsl, slab, 0)
        plsc.subcore_barrier()
        pltpu.sync_copy(agg_sh.at[pl.ds(s * _RPT, _RPT)],
                        out_hbm.at[c, pl.ds(s * _RPT, _RPT)])

    return k, che


def _bn_cols(hp, gamma, beta):
    mean = jnp.sum(hp, axis=0, keepdims=True) * (1.0 / _N)
    var = jnp.sum(hp * hp, axis=0, keepdims=True) * (1.0 / _N) - mean * mean
    scale = gamma * lax.rsqrt(var + _EPS)
    return hp * scale + (beta - mean * scale)


def _pre1(x, W):
    def body(x_ref, W_ref, y_ref):
        y_ref[...] = jnp.dot(x_ref[...], W_ref[...], precision=_PREC,
                             preferred_element_type=jnp.float32)

    return pl.pallas_call(
        body,
        out_shape=jax.ShapeDtypeStruct((_N, _H), jnp.float32),
    )(x, W)


def _dense(y, a0, a1, p, W1n, res):
    has_res = res is not None

    def body(*refs):
        (y_ref, a0_ref, a1_ref, b1_ref, g1_ref, be1_ref, W2_ref, b2_ref,
         g2_ref, be2_ref, W1n_ref) = refs[:11]
        rest = refs[11:]
        if has_res:
            r_ref = rest[0]
            rest = rest[1:]
        h_ref, yn_ref = rest
        hp = y_ref[...] + a0_ref[...] + a1_ref[...] + b1_ref[...]
        r1 = jnp.maximum(_bn_cols(hp, g1_ref[...], be1_ref[...]), 0.0)
        hp2 = jnp.dot(r1, W2_ref[...], precision=_PREC,
                      preferred_element_type=jnp.float32) + b2_ref[...]
        h = jnp.maximum(_bn_cols(hp2, g2_ref[...], be2_ref[...]), 0.0)
        if has_res:
            h = h + r_ref[...]
        h_ref[...] = h
        yn_ref[...] = jnp.dot(h, W1n_ref[...], precision=_PREC,
                              preferred_element_type=jnp.float32)

    args = [y, a0, a1, p['b1'].reshape(1, -1), p['g1'].reshape(1, -1),
            p['be1'].reshape(1, -1), p['W2'], p['b2'].reshape(1, -1),
            p['g2'].reshape(1, -1), p['be2'].reshape(1, -1), W1n]
    if has_res:
        args.append(res)

    return pl.pallas_call(
        body,
        out_shape=[
            jax.ShapeDtypeStruct((_N, _H), jnp.float32),
            jax.ShapeDtypeStruct((_N, _H), jnp.float32),
        ],
    )(*args)


def _final(y, a0, a1, p, res, batch_f, hp):
    names = ['head_color', 'head_size', 'head_ground', 'head_struct']
    douts = [16, 16, 8, 32]

    def body(*refs):
        (y_ref, a0_ref, a1_ref, b1_ref, g1_ref, be1_ref, W2_ref, b2_ref,
         g2_ref, be2_ref, r_ref, bat_ref) = refs[:12]
        hrefs = refs[12:12 + 4 * len(names)]
        outs = refs[12 + 4 * len(names):]
        hp_ = y_ref[...] + a0_ref[...] + a1_ref[...] + b1_ref[...]
        r1 = jnp.maximum(_bn_cols(hp_, g1_ref[...], be1_ref[...]), 0.0)
        hp2 = jnp.dot(r1, W2_ref[...], precision=_PREC,
                      preferred_element_type=jnp.float32) + b2_ref[...]
        h = jnp.maximum(_bn_cols(hp2, g2_ref[...], be2_ref[...]), 0.0)
        h = h + r_ref[...]
        seg = lax.broadcasted_iota(jnp.int32, (_N, _G), 1).astype(jnp.float32)
        onehot = jnp.where(bat_ref[...] == seg, 1.0, 0.0)
        g = lax.dot_general(onehot, h, (((0,), (0,)), ((), ())),
                            precision=_PREC,
                            preferred_element_type=jnp.float32)
        for n in range(len(names)):
            W1, b1, W2, b2 = hrefs[4 * n:4 * (n + 1)]
            rr = jnp.maximum(
                jnp.dot(g, W1[...], precision=_PREC,
                        preferred_element_type=jnp.float32) + b1[...], 0.0)
            z = jnp.dot(rr, W2[...], precision=_PREC,
                        preferred_element_type=jnp.float32) + b2[...]
            nrm = jnp.sqrt(jnp.sum(z * z, axis=1, keepdims=True))
            outs[n][...] = z / jnp.maximum(nrm, 1e-12)

    args = [y, a0, a1, p['b1'].reshape(1, -1), p['g1'].reshape(1, -1),
            p['be1'].reshape(1, -1), p['W2'], p['b2'].reshape(1, -1),
            p['g2'].reshape(1, -1), p['be2'].reshape(1, -1), res, batch_f]
    for n in names:
        q = hp[n]
        args += [q['W1'], q['b1'].reshape(1, -1), q['W2'],
                 q['b2'].reshape(1, -1)]

    return pl.pallas_call(
        body,
        out_shape=[jax.ShapeDtypeStruct((_G, d), jnp.float32) for d in douts],
    )(*args)


def kernel(x, edge_index, batch, params):
    src = edge_index[0]
    dst = edge_index[1]
    npad = _EPW - _E // _NW
    dummy_dst = jnp.broadcast_to(
        _DUMMY + (jnp.arange(npad, dtype=jnp.int32) % (_NP - _N)),
        (_NW, npad))
    pad_src = jnp.concatenate(
        [src.reshape(_NW, _E // _NW),
         jnp.zeros((_NW, npad), jnp.int32)], axis=1)
    pad_dst = jnp.concatenate(
        [dst.reshape(_NW, _E // _NW), dummy_dst], axis=1)
    zblk = jnp.zeros((_RPT, _H), jnp.float32)
    batch_f = batch.astype(jnp.float32).reshape(_N, 1)

    seg, che = _seg_sum_sc(_H)
    srcp = pad_src.reshape(_NW, _EPW // che, che)
    dstp = pad_dst.reshape(_NW, _EPW // che, che)

    def agg2(y):
        agg = seg(y, srcp, dstp, zblk)
        a0 = lax.slice(agg, (0, 0, 0), (1, _N, _H)).reshape(_N, _H)
        a1 = lax.slice(agg, (1, 0, 0), (2, _N, _H)).reshape(_N, _H)
        return a0, a1

    y1 = _pre1(x, params['conv1']['W1'])
    a0, a1 = agg2(y1)
    h1, y2 = _dense(y1, a0, a1, params['conv1'], params['conv2']['W1'], None)
    a0, a1 = agg2(y2)
    h2, y3 = _dense(y2, a0, a1, params['conv2'], params['conv3']['W1'], h1)
    a0, a1 = agg2(y3)
    return _final(y3, a0, a1, params['conv3'], h2, batch_f, params)
